# baseline (speedup 1.0000x reference)
import functools

import jax
import jax.numpy as jnp
from jax import lax
from jax.experimental import pallas as pl
from jax.experimental.pallas import tpu as pltpu
from jax.experimental.pallas import tpu_sc as plsc

_BETA = 0.25
_G = 8


def _argmax_kernel(rn_ref, en_ref, rprev_ref, qprev_ref, idx_ref, qp_ref, *,
                   vocab, vt, first):
    tn = rn_ref.shape[0]
    nv = vt // _G

    @pl.when(pl.program_id(0) == 0)
    def _init():
        qp_ref[...] = jnp.zeros_like(qp_ref)

    if not first:
        q = qprev_ref[:, :rprev_ref.shape[1]]
        rp = rprev_ref[...]
        diff = q - rp
        part = jnp.sum(diff * diff)
        mask = ((lax.broadcasted_iota(jnp.int32, (8, 128), 0) == 0)
                & (lax.broadcasted_iota(jnp.int32, (8, 128), 1) == 0))
        qp_ref[...] = qp_ref[...] + jnp.where(mask, part, 0.0)

    rnb = rn_ref[...].astype(jnp.bfloat16)

    def tile_body(t, carry):
        M, fb, la, ha = carry
        en_t = en_ref[pl.ds(t * vt, vt), :].astype(jnp.bfloat16)
        s = lax.dot_general(rnb, en_t, (((1,), (1,)), ((), ())),
                            preferred_element_type=jnp.float32)
        b = s.astype(jnp.bfloat16).astype(jnp.float32)
        tm = jnp.max(b, axis=1, keepdims=True)
        iota = lax.broadcasted_iota(jnp.int32, (tn, vt), 1)
        bucket = b == tm
        abv = bucket & (s > tm)
        t_fb = jnp.min(jnp.where(bucket, iota, vt), axis=1, keepdims=True) + t * vt
        t_lar = jnp.max(jnp.where(abv, iota, -1), axis=1, keepdims=True)
        t_ha = (t_lar >= 0).astype(jnp.int32)
        t_la = t_lar + t * vt
        newmax = tm > M
        same = tm == M
        M2 = jnp.where(newmax, tm, M)
        fb2 = jnp.where(newmax, t_fb, fb)
        la2 = jnp.where(newmax | (same & (t_ha > 0)), t_la, la)
        ha2 = jnp.where(newmax, t_ha, jnp.maximum(ha, same.astype(jnp.int32) * t_ha))
        return M2, fb2, la2, ha2

    init = (jnp.full((tn, 1), -jnp.inf, dtype=jnp.float32),
            jnp.zeros((tn, 1), dtype=jnp.int32),
            jnp.zeros((tn, 1), dtype=jnp.int32),
            jnp.zeros((tn, 1), dtype=jnp.int32))
    M, fb, la, ha = lax.fori_loop(0, vocab // vt, tile_body, init)
    win = jnp.where(ha > 0, la, fb)
    idx_ref[...] = jnp.broadcast_to(win[:, 0][None, :], idx_ref.shape)


def _epilogue_kernel(z_ref, r_ref, q_ref, zq_ref, qp_ref):
    @pl.when(pl.program_id(0) == 0)
    def _init():
        qp_ref[...] = jnp.zeros_like(qp_ref)

    z = z_ref[...]
    r = r_ref[...]
    q = q_ref[:, :z_ref.shape[1]]
    zq_ref[...] = z - (r - q)
    diff = q - r
    part = jnp.sum(diff * diff)
    mask = ((lax.broadcasted_iota(jnp.int32, (8, 128), 0) == 0)
            & (lax.broadcasted_iota(jnp.int32, (8, 128), 1) == 0))
    qp_ref[...] = qp_ref[...] + jnp.where(mask, part, 0.0)


def _tc_argmax(rn, en, r_prev, q_prev, *, first, tn=512, vt=512):
    n, c = rn.shape
    v = en.shape[0]
    kern = functools.partial(_argmax_kernel, vocab=v, vt=vt, first=first)
    idx, qp = pl.pallas_call(
        kern,
        grid=(n // tn,),
        in_specs=[
            pl.BlockSpec((tn, c), lambda i: (i, 0)),
            pl.BlockSpec((v, c), lambda i: (0, 0)),
            pl.BlockSpec((tn, c), lambda i: (i, 0)),
            pl.BlockSpec((tn, 128), lambda i: (i, 0)),
        ],
        out_specs=[
            pl.BlockSpec((8, tn), lambda i: (0, i)),
            pl.BlockSpec((8, 128), lambda i: (0, 0)),
        ],
        out_shape=[
            jax.ShapeDtypeStruct((8, n), jnp.int32),
            jax.ShapeDtypeStruct((8, 128), jnp.float32),
        ],
    )(rn, en, r_prev, q_prev)
    return idx[0], jnp.sum(qp)


def _make_sc_gather(v, n):
    rw = 128
    info = plsc.get_sparse_core_info()
    nw = info.num_cores * info.num_subcores
    b_per_w = n // nw
    n_chunks = b_per_w // 128
    mesh = plsc.VectorSubcoreMesh(core_axis_name="c", subcore_axis_name="s")

    @functools.partial(
        pl.kernel, mesh=mesh,
        out_type=jax.ShapeDtypeStruct((n, rw), jnp.float32),
        scratch_types=[
            pltpu.VMEM((b_per_w,), jnp.int32),
            pltpu.VMEM((b_per_w, rw), jnp.float32),
            pltpu.SemaphoreType.DMA,
        ],
    )
    def gather(table_hbm, idx_hbm, out_hbm, idx_v, rows_v, sem):
        wid = lax.axis_index("s") * info.num_cores + lax.axis_index("c")
        base = wid * b_per_w
        pltpu.sync_copy(idx_hbm.at[pl.ds(base, b_per_w)], idx_v)
        copies = []
        for j in range(n_chunks):
            sl = pl.ds(j * 128, 128)
            copies.append(pltpu.async_copy(
                table_hbm.at[idx_v.at[sl]], rows_v.at[sl, :], sem))
        for cp in copies:
            cp.wait()
        pltpu.sync_copy(rows_v, out_hbm.at[pl.ds(base, b_per_w)])

    return gather


def _to_tok(x_nchw):
    B, C, H, W = x_nchw.shape
    return jnp.transpose(x_nchw, (0, 2, 3, 1)).reshape(B * H * W, C)


@jax.jit
def kernel(z, codebooks):
    B, C, H, W = z.shape
    L, V, _ = codebooks.shape
    N = B * H * W

    sc_gather = _make_sc_gather(V, N)
    cb_pad = jnp.pad(codebooks, ((0, 0), (0, 0), (0, 128 - C)))

    r_nchw = z
    r_prev_tok = None
    q = jnp.zeros((N, 128), jnp.float32)
    qp_total = jnp.zeros((), jnp.float32)
    idx_levels = []
    for l in range(L):
        nrm = jnp.clip(jnp.linalg.norm(r_nchw, axis=1, keepdims=True), 1e-12)
        rn = _to_tok(r_nchw / nrm)
        emb = codebooks[l]
        en = emb / jnp.clip(jnp.linalg.norm(emb, axis=1, keepdims=True), 1e-12)
        r_tok = _to_tok(r_nchw)
        idx, qp = _tc_argmax(rn, en,
                             r_prev_tok if l else r_tok, q, first=(l == 0))
        qp_total = qp_total + qp
        idx_levels.append(idx)
        q = sc_gather(cb_pad[l], idx)
        q_nchw = jnp.transpose(q[:, :C].reshape(B, H, W, C), (0, 3, 1, 2))
        r_prev_tok = r_tok
        r_nchw = r_nchw - q_nchw

    zq_tok, qp_last = pl.pallas_call(
        _epilogue_kernel,
        grid=(N // 512,),
        in_specs=[
            pl.BlockSpec((512, C), lambda i: (i, 0)),
            pl.BlockSpec((512, C), lambda i: (i, 0)),
            pl.BlockSpec((512, 128), lambda i: (i, 0)),
        ],
        out_specs=[
            pl.BlockSpec((512, C), lambda i: (i, 0)),
            pl.BlockSpec((8, 128), lambda i: (0, 0)),
        ],
        out_shape=[
            jax.ShapeDtypeStruct((N, C), jnp.float32),
            jax.ShapeDtypeStruct((8, 128), jnp.float32),
        ],
    )(_to_tok(z), r_prev_tok, q)
    qp_total = qp_total + jnp.sum(qp_last)

    z_q = jnp.transpose(zq_tok.reshape(B, H, W, C), (0, 3, 1, 2))
    all_idx = jnp.stack(idx_levels, axis=0).reshape(L, B, H, W)
    qloss = _BETA * qp_total / (N * C)
    return z_q, all_idx, qloss
